# Initial kernel scaffold; baseline (speedup 1.0000x reference)
import jax
import jax.numpy as jnp
from jax.experimental import pallas as pl


def kernel(x, emb):
    raise NotImplementedError("write your pallas kernel here")


# speedup vs baseline: 1.1083x; 1.1083x over previous
import functools

import jax
import jax.numpy as jnp
from jax import lax
from jax.experimental import pallas as pl
from jax.experimental.pallas import tpu as pltpu
from jax.experimental.pallas import tpu_sc as plsc

EMB_DIM = 32

_info = plsc.get_sparse_core_info()
_NC, _NS = _info.num_cores, _info.num_subcores
_NW = _NC * _NS

_CHUNK = 2560


def _gather_body(x_hbm, emb_hbm, out_hbm, idx_v, rows_v, sem, *, b_per_w, chunk):
    wid = lax.axis_index("s") * _NC + lax.axis_index("c")
    base_w = wid * b_per_w
    n_chunks = b_per_w // chunk
    for i in range(n_chunks):
        base = base_w + i * chunk
        pltpu.sync_copy(x_hbm.at[pl.ds(base, chunk)], idx_v)
        pltpu.async_copy(emb_hbm.at[idx_v], rows_v, sem).wait()
        pltpu.sync_copy(rows_v, out_hbm.at[pl.ds(base, chunk)])


def kernel(x, emb):
    B = x.shape[0] * x.shape[1]
    idx = x.reshape(B).astype(jnp.int32)
    b_per_w = B // _NW
    chunk = _CHUNK

    mesh = plsc.VectorSubcoreMesh(core_axis_name="c", subcore_axis_name="s")
    run = pl.kernel(
        functools.partial(_gather_body, b_per_w=b_per_w, chunk=chunk),
        out_type=jax.ShapeDtypeStruct((B, EMB_DIM), jnp.float32),
        mesh=mesh,
        scratch_types=[
            pltpu.VMEM((chunk,), jnp.int32),
            pltpu.VMEM((chunk, EMB_DIM), jnp.float32),
            pltpu.SemaphoreType.DMA,
        ],
        compiler_params=pltpu.CompilerParams(use_tc_tiling_on_sc=False),
    )
    out = run(idx, emb)
    return out.reshape(x.shape[0], x.shape[1], EMB_DIM)


# speedup vs baseline: 1.1093x; 1.0009x over previous
import functools

import jax
import jax.numpy as jnp
from jax import lax
from jax.experimental import pallas as pl
from jax.experimental.pallas import tpu as pltpu
from jax.experimental.pallas import tpu_sc as plsc

EMB_DIM = 32

_info = plsc.get_sparse_core_info()
_NC, _NS = _info.num_cores, _info.num_subcores
_NW = _NC * _NS

_CHUNK = 1600


def _gather_body(x_hbm, emb_hbm, out_hbm,
                 idx0, idx1, rows0, rows1,
                 si0, si1, sg, sw0, sw1, *, b_per_w, chunk):
    wid = lax.axis_index("s") * _NC + lax.axis_index("c")
    base_w = wid * b_per_w
    n = b_per_w // chunk
    idx = [idx0, idx1]
    rows = [rows0, rows1]
    si = [si0, si1]
    sw = [sw0, sw1]
    h_w = [None, None]

    h_i = pltpu.async_copy(x_hbm.at[pl.ds(base_w, chunk)], idx[0], si[0])
    for i in range(n):
        b = i & 1
        if i + 1 < n:
            nh_i = pltpu.async_copy(
                x_hbm.at[pl.ds(base_w + (i + 1) * chunk, chunk)],
                idx[1 - b], si[1 - b])
        h_i.wait()
        if h_w[b] is not None:
            h_w[b].wait()
        pltpu.async_copy(emb_hbm.at[idx[b]], rows[b], sg).wait()
        h_w[b] = pltpu.async_copy(
            rows[b], out_hbm.at[pl.ds(base_w + i * chunk, chunk)], sw[b])
        if i + 1 < n:
            h_i = nh_i
    h_w[(n - 1) & 1].wait()
    if n >= 2:
        h_w[n & 1].wait()


def kernel(x, emb):
    B = x.shape[0] * x.shape[1]
    idx = x.reshape(B).astype(jnp.int32)
    b_per_w = B // _NW
    chunk = _CHUNK

    mesh = plsc.VectorSubcoreMesh(core_axis_name="c", subcore_axis_name="s")
    run = pl.kernel(
        functools.partial(_gather_body, b_per_w=b_per_w, chunk=chunk),
        out_type=jax.ShapeDtypeStruct((B, EMB_DIM), jnp.float32),
        mesh=mesh,
        scratch_types=[
            pltpu.VMEM((chunk,), jnp.int32),
            pltpu.VMEM((chunk,), jnp.int32),
            pltpu.VMEM((chunk, EMB_DIM), jnp.float32),
            pltpu.VMEM((chunk, EMB_DIM), jnp.float32),
            pltpu.SemaphoreType.DMA,
            pltpu.SemaphoreType.DMA,
            pltpu.SemaphoreType.DMA,
            pltpu.SemaphoreType.DMA,
            pltpu.SemaphoreType.DMA,
        ],
        compiler_params=pltpu.CompilerParams(use_tc_tiling_on_sc=False),
    )
    out = run(idx, emb)
    return out.reshape(x.shape[0], x.shape[1], EMB_DIM)
